# baseline (speedup 1.0000x reference)
import math

import jax
import jax.numpy as jnp
from jax import lax
from jax.experimental import pallas as pl
from jax.experimental.pallas import tpu as pltpu
from jax.experimental.pallas import tpu_sc as plsc

L = 200
B = 4096
DIM = 64
SCALE = math.sqrt(DIM)

_info = plsc.get_sparse_core_info()
NC, NS = _info.num_cores, _info.num_subcores
NW = NC * NS
CH = B // NW
NBUF = 4
ROUNDS = L // NBUF
DT = DIM // 8
OUT_ROWS = L * DT * (B // 128)


def _sc_body(idx_hbm, w_hbm, pe_hbm, out_hbm, pe_v, idx_v, rin, tout, *sems):
    gsem = sems[:NBUF]
    ssem = sems[NBUF:]
    wid = lax.axis_index("s") * NC + lax.axis_index("c")
    col = wid * CH
    pltpu.sync_copy(pe_hbm, pe_v)
    pltpu.sync_copy(idx_hbm.at[:, pl.ds(col, CH)], idx_v)

    lane = lax.iota(jnp.int32, 16)
    scat_idx = [lane * CH + j * 16 * CH for j in range(DIM // 16)]

    def fire_gather(l, b):
        pltpu.async_copy(w_hbm.at[idx_v.at[l]], rin.at[b], gsem[b])

    def wait_gather(l, b):
        pltpu.make_async_copy(w_hbm.at[idx_v.at[l]], rin.at[b], gsem[b]).wait()

    def fire_store(l, b):
        for dt in range(DT):
            pltpu.async_copy(
                tout.at[b, pl.ds(dt * 1024, 1024)],
                out_hbm.at[l * (DT * 32) + dt * 32 + wid],
                ssem[b],
            )

    def wait_store(l, b):
        for dt in range(DT):
            pltpu.make_async_copy(
                tout.at[b, pl.ds(dt * 1024, 1024)],
                out_hbm.at[l * (DT * 32) + dt * 32 + wid],
                ssem[b],
            ).wait()

    def tfma(l, b):
        pe_regs = [pe_v[l, pl.ds(16 * j, 16)] for j in range(DIM // 16)]

        @plsc.parallel_loop(0, CH, step=1, unroll=2)
        def _(r):
            rsplat = lax.broadcast(r, (16,))
            for j in range(DIM // 16):
                v = rin[b, r, pl.ds(16 * j, 16)]
                plsc.store_scatter(
                    tout.at[b], [scat_idx[j] + rsplat], v * SCALE + pe_regs[j]
                )

    def step(l, b, first, fire_next):
        wait_gather(l, b)
        if not first:
            wait_store(l - NBUF, b)
        tfma(l, b)
        fire_store(l, b)
        if fire_next:
            fire_gather(l + NBUF, b)

    for b in range(NBUF):
        fire_gather(b, b)
    for b in range(NBUF):
        step(b, b, first=True, fire_next=True)

    def round_body(mc, _):
        for b in range(NBUF):
            step(mc * NBUF + b, b, first=False, fire_next=True)
        return 0

    lax.fori_loop(1, ROUNDS - 1, round_body, 0)

    last = (ROUNDS - 1) * NBUF
    for b in range(NBUF):
        step(last + b, b, first=False, fire_next=False)
    for b in range(NBUF):
        wait_store(last + b, b)


@jax.jit
def _embed(idx, W, pe_s):
    mesh = plsc.VectorSubcoreMesh(core_axis_name="c", subcore_axis_name="s")
    f = pl.kernel(
        _sc_body,
        out_type=jax.ShapeDtypeStruct((OUT_ROWS, 1024), jnp.float32),
        mesh=mesh,
        scratch_types=[
            pltpu.VMEM((L, DIM), jnp.float32),
            pltpu.VMEM((L, CH), jnp.int32),
            pltpu.VMEM((NBUF, CH, DIM), jnp.float32),
            pltpu.VMEM((NBUF, DIM * CH), jnp.float32),
        ]
        + [pltpu.SemaphoreType.DMA] * (2 * NBUF),
        compiler_params=pltpu.CompilerParams(
            use_tc_tiling_on_sc=False, needs_layout_passes=False
        ),
    )
    return f(idx, W, pe_s)


def kernel(source, W, pe):
    idx = source.reshape(L, B)
    pe_s = pe[:L, 0, :]
    out5 = _embed(idx, W, pe_s).reshape(L, DT, B // 128, 8, 128)
    return out5.transpose(0, 2, 4, 1, 3).reshape(L, B, DIM)


# speedup vs baseline: 1.5552x; 1.5552x over previous
import math

import jax
import jax.numpy as jnp
from jax import lax
from jax.experimental import pallas as pl
from jax.experimental.pallas import tpu as pltpu
from jax.experimental.pallas import tpu_sc as plsc

L = 200
B = 4096
DIM = 64
N = L * B
VOCAB = 1000000
SCALE = math.sqrt(DIM)

_info = plsc.get_sparse_core_info()
NC, NS = _info.num_cores, _info.num_subcores
NW = NC * NS
PERW = N // NW
CHK = 256
NCHK = PERW // CHK
NB = 2
NI = 3
OUT_ROWS = L * 8 * (B // 128)


def _sc_body(idx_hbm, w_hbm, pe_hbm, out_hbm, pe_v, idxr, rin, tskew, tout, *sems):
    gsem = sems[0:NB]
    ssem = sems[NB]
    isem = sems[NB + 1]
    lane = lax.iota(jnp.int32, 16)
    wid = lax.axis_index("s") * NC + lax.axis_index("c")
    base_w = wid * PERW
    pltpu.sync_copy(pe_hbm, pe_v)

    def idx_pair(c):
        src = idx_hbm.at[pl.ds(base_w // CHK + c, 1)]
        return src, idxr.at[pl.ds(c % NI, 1)], isem

    def fire_idx(c):
        pltpu.async_copy(*idx_pair(c))

    def wait_idx(c):
        pltpu.make_async_copy(*idx_pair(c)).wait()

    def fire_gather(c, b):
        pltpu.async_copy(w_hbm.at[idxr.at[c % NI]], rin.at[b], gsem[b])

    def wait_gather(c, b):
        pltpu.make_async_copy(
            w_hbm.at[idxr.at[c % NI]], rin.at[b], gsem[b]
        ).wait()

    def store_list(c, b):
        pos = base_w + c * CHK
        row0 = (pos // B) * 256 + (pos % B) // 128
        out = []
        for dt in range(8):
            for bt in range(CHK // 128):
                out.append((
                    tout.at[pl.ds((dt * (CHK // 128) + bt) * 1024, 1024)],
                    out_hbm.at[pl.ds((row0 + dt * 32 + bt) * 1024, 1024)],
                    ssem,
                ))
        return out

    def fire_store(c, b):
        for t in store_list(c, b):
            pltpu.async_copy(*t)

    def wait_store(c, b):
        for t in store_list(c, b):
            pltpu.make_async_copy(*t).wait()

    def tfma(c, b):
        pos = base_w + c * CHK
        l = pos // B
        pe_regs = [pe_v[l, pl.ds(16 * j, 16)] for j in range(DIM // 16)]
        lane256 = [lane * CHK + 16 * j * CHK for j in range(DIM // 16)]

        @plsc.parallel_loop(0, CHK, step=1, unroll=2)
        def _(r):
            rv = lane + lax.broadcast(r, (16,))
            for j in range(DIM // 16):
                v = rin[b, r, pl.ds(16 * j, 16)] * SCALE + pe_regs[j]
                sk = lane256[j] + ((rv + 16 * j) & (CHK - 1))
                plsc.store_scatter(tskew, [sk], v)

        @plsc.parallel_loop(0, DIM, step=1, unroll=2)
        def _(d):
            lv = lane + lax.broadcast(d, (16,))
            dbase = lax.broadcast(d * CHK, (16,))
            tbase = (d // 8) * ((CHK // 128) * 1024) + (d % 8) * 128
            for bt in range(CHK // 128):
                for k in range(8):
                    sk = dbase + ((lv + (bt * 128 + 16 * k)) & (CHK - 1))
                    v = plsc.load_gather(tskew, [sk])
                    tout[pl.ds(tbase + bt * 1024 + 16 * k, 16)] = v

    def step(c, b, *, wait_s, fire_g, fire_i):
        wait_gather(c, b)
        if wait_s:
            wait_store(c - 1, 0)
        tfma(c, b)
        fire_store(c, b)
        if fire_g:
            wait_idx(c + NB)
            fire_gather(c + NB, b)
        if fire_i:
            fire_idx(c + NI)

    for c in range(NI):
        fire_idx(c)
    wait_idx(0)
    fire_gather(0, 0)
    wait_idx(1)
    fire_gather(1, 1)
    step(0, 0, wait_s=False, fire_g=True, fire_i=True)
    step(1, 1, wait_s=True, fire_g=True, fire_i=True)

    def round_body(mc, _):
        for b in range(NB):
            step(mc * NB + b, b, wait_s=True, fire_g=True, fire_i=True)
        return 0

    lax.fori_loop(1, (NCHK - 4) // NB, round_body, 0)

    for c in range(NCHK - 4, NCHK):
        step(
            c,
            c % NB,
            wait_s=True,
            fire_g=c + NB < NCHK,
            fire_i=c + NI < NCHK,
        )
    wait_store(NCHK - 1, 0)


@jax.jit
def _embed(idx, W, pe_s):
    mesh = plsc.VectorSubcoreMesh(core_axis_name="c", subcore_axis_name="s")
    f = pl.kernel(
        _sc_body,
        out_type=jax.ShapeDtypeStruct((OUT_ROWS * 1024,), jnp.float32),
        mesh=mesh,
        scratch_types=[
            pltpu.VMEM((L, DIM), jnp.float32),
            pltpu.VMEM((NI, CHK), jnp.int32),
            pltpu.VMEM((NB, CHK, DIM), jnp.float32),
            pltpu.VMEM((DIM * CHK,), jnp.float32),
            pltpu.VMEM((8 * (CHK // 128) * 1024,), jnp.float32),
        ]
        + [pltpu.SemaphoreType.DMA] * (NB + 2),
        compiler_params=pltpu.CompilerParams(
            use_tc_tiling_on_sc=False, needs_layout_passes=False
        ),
    )
    return f(idx, W, pe_s)


def kernel(source, W, pe):
    idx = source.reshape(N // CHK, CHK)
    pe_s = pe[:L, 0, :]
    out5 = _embed(idx, W, pe_s).reshape(L, 8, B // 128, 8, 128)
    return out5.transpose(0, 2, 4, 1, 3).reshape(L, B, DIM)


# speedup vs baseline: 1.6543x; 1.0637x over previous
import math

import jax
import jax.numpy as jnp
from jax import lax
from jax.experimental import pallas as pl
from jax.experimental.pallas import tpu as pltpu
from jax.experimental.pallas import tpu_sc as plsc

L = 200
B = 4096
DIM = 64
N = L * B
SCALE = math.sqrt(DIM)

_info = plsc.get_sparse_core_info()
NC, NS = _info.num_cores, _info.num_subcores
NW = NC * NS
PERW = N // NW
CHK = 512
HC = 256
NCHK = PERW // CHK
NB = 2
NI = 3
OUT_ROWS = L * 8 * (B // 128)


def _sc_body(idx_hbm, w_hbm, pe_hbm, out_hbm, pe_v, idxr, rin, tskew, tout, *sems):
    gsem = sems[0:NB]
    ssem = sems[NB : NB + 2]
    isem = sems[NB + 2]
    lane = lax.iota(jnp.int32, 16)
    wid = lax.axis_index("s") * NC + lax.axis_index("c")
    base_w = wid * PERW
    pltpu.sync_copy(pe_hbm, pe_v)

    def idx_pair(c):
        src = idx_hbm.at[pl.ds(base_w // CHK + c, 1)]
        return src, idxr.at[pl.ds(c % NI, 1)], isem

    def fire_idx(c):
        pltpu.async_copy(*idx_pair(c))

    def wait_idx(c):
        pltpu.make_async_copy(*idx_pair(c)).wait()

    def fire_gather(c, b):
        pltpu.async_copy(w_hbm.at[idxr.at[c % NI]], rin.at[b], gsem[b])

    def wait_gather(c, b):
        pltpu.make_async_copy(w_hbm.at[idxr.at[c % NI]], rin.at[b], gsem[b]).wait()

    def store_list(c, h):
        pos = base_w + c * CHK + h * HC
        row0 = (pos // B) * 256 + (pos % B) // 128
        out = []
        for dt in range(8):
            for bt in range(HC // 128):
                out.append((
                    tout.at[h, pl.ds((dt * (HC // 128) + bt) * 1024, 1024)],
                    out_hbm.at[pl.ds((row0 + dt * 32 + bt) * 1024, 1024)],
                    ssem[h],
                ))
        return out

    def fire_store(c, h):
        for t in store_list(c, h):
            pltpu.async_copy(*t)

    def wait_store(c, h):
        for t in store_list(c, h):
            pltpu.make_async_copy(*t).wait()

    def tfma(c, b, h):
        pos = base_w + c * CHK + h * HC
        l = pos // B
        pe_regs = [pe_v[l, pl.ds(16 * j, 16)] for j in range(DIM // 16)]
        lane_hc = [lane * HC + 16 * j * HC for j in range(DIM // 16)]

        @plsc.parallel_loop(0, HC, step=1, unroll=2)
        def _(r):
            rv = lane + lax.broadcast(r, (16,))
            for j in range(DIM // 16):
                v = rin[b, h * HC + r, pl.ds(16 * j, 16)] * SCALE + pe_regs[j]
                sk = lane_hc[j] + ((rv + 16 * j) & (HC - 1))
                plsc.store_scatter(tskew, [sk], v)

        @plsc.parallel_loop(0, DIM, step=1, unroll=2)
        def _(d):
            lv = lane + lax.broadcast(d, (16,))
            dbase = lax.broadcast(d * HC, (16,))
            tbase = (d // 8) * ((HC // 128) * 1024) + (d % 8) * 128
            for bt in range(HC // 128):
                for k in range(8):
                    sk = dbase + ((lv + (bt * 128 + 16 * k)) & (HC - 1))
                    v = plsc.load_gather(tskew, [sk])
                    tout[h, pl.ds(tbase + bt * 1024 + 16 * k, 16)] = v

    def step(c, b, *, wait_s, fire_g, fire_i):
        wait_gather(c, b)
        for h in range(2):
            if wait_s:
                wait_store(c - 1, h)
            tfma(c, b, h)
            fire_store(c, h)
        if fire_g:
            wait_idx(c + NB)
            fire_gather(c + NB, b)
        if fire_i:
            fire_idx(c + NI)

    for c in range(NI):
        fire_idx(c)
    wait_idx(0)
    fire_gather(0, 0)
    wait_idx(1)
    fire_gather(1, 1)
    step(0, 0, wait_s=False, fire_g=True, fire_i=True)
    step(1, 1, wait_s=True, fire_g=True, fire_i=True)

    def round_body(mc, _):
        for b in range(NB):
            step(mc * NB + b, b, wait_s=True, fire_g=True, fire_i=True)
        return 0

    lax.fori_loop(1, (NCHK - 4) // NB, round_body, 0)

    for c in range(NCHK - 4, NCHK):
        step(
            c,
            c % NB,
            wait_s=True,
            fire_g=c + NB < NCHK,
            fire_i=c + NI < NCHK,
        )
    for h in range(2):
        wait_store(NCHK - 1, h)


@jax.jit
def _embed(idx, W, pe_s):
    mesh = plsc.VectorSubcoreMesh(core_axis_name="c", subcore_axis_name="s")
    f = pl.kernel(
        _sc_body,
        out_type=jax.ShapeDtypeStruct((OUT_ROWS * 1024,), jnp.float32),
        mesh=mesh,
        scratch_types=[
            pltpu.VMEM((L, DIM), jnp.float32),
            pltpu.VMEM((NI, CHK), jnp.int32),
            pltpu.VMEM((NB, CHK, DIM), jnp.float32),
            pltpu.VMEM((DIM * HC,), jnp.float32),
            pltpu.VMEM((2, 8 * (HC // 128) * 1024), jnp.float32),
        ]
        + [pltpu.SemaphoreType.DMA] * (NB + 3),
        compiler_params=pltpu.CompilerParams(
            use_tc_tiling_on_sc=False, needs_layout_passes=False
        ),
    )
    return f(idx, W, pe_s)


def kernel(source, W, pe):
    idx = source.reshape(N // CHK, CHK)
    pe_s = pe[:L, 0, :]
    out5 = _embed(idx, W, pe_s).reshape(L, 8, B // 128, 8, 128)
    return out5.transpose(0, 2, 4, 1, 3).reshape(L, B, DIM)
